# baseline (speedup 1.0000x reference)
import functools

import jax
import jax.numpy as jnp
from jax import lax
from jax.experimental import pallas as pl
from jax.experimental.pallas import tpu as pltpu
from jax.experimental.pallas import tpu_sc as plsc

N = 10000
D = 128
H = 128
G4 = 4 * H
CO = 64
E = 320000

NC = 2
NS = 16
NW = NC * NS
EPW = E // NW
KCH = 80
NCHUNK = EPW // KCH
NPAIR = NCHUNK // 2
KDEG = 80
NDCH = EPW // KDEG
NDPAIR = NDCH // 2
HH = H // 2
RA = 624
TAIL = N - NS * RA
TAIL_OFF = NS * RA

def _sc_degrees_body(src_hbm, dst_hbm, z_hbm, mska_hbm, mskb_hbm, out_hbm,
                     siA, diA, siB, diB, mska_v, mskb_v, acc_sh, semA, semB):
    c = lax.axis_index("c")
    s = lax.axis_index("s")
    wid = s * NC + c
    base = wid * EPW
    pltpu.sync_copy(mska_hbm, mska_v)
    pltpu.sync_copy(mskb_hbm, mskb_v)
    pltpu.sync_copy(z_hbm.at[pl.ds(0, RA)], acc_sh.at[pl.ds(s * RA, RA)])

    @pl.when(s == NS - 1)
    def _ztail():
        pltpu.sync_copy(z_hbm.at[pl.ds(0, TAIL)],
                        acc_sh.at[pl.ds(TAIL_OFF, TAIL)])

    plsc.subcore_barrier()

    pltpu.async_copy(src_hbm.at[pl.ds(base, KDEG)], siA, semA)
    pltpu.async_copy(dst_hbm.at[pl.ds(base, KDEG)], diA, semA)

    def body(it, carry):
        offb = base + (2 * it + 1) * KDEG
        pltpu.async_copy(src_hbm.at[pl.ds(offb, KDEG)], siB, semB)
        pltpu.async_copy(dst_hbm.at[pl.ds(offb, KDEG)], diB, semB)

        pltpu.make_async_copy(src_hbm.at[pl.ds(base, KDEG)], siA, semA).wait()
        pltpu.make_async_copy(dst_hbm.at[pl.ds(base, KDEG)], diA, semA).wait()
        pltpu.sync_copy(mska_v, acc_sh.at[siA], add=True)
        pltpu.sync_copy(mskb_v, acc_sh.at[diA], add=True)

        @pl.when(it < NDPAIR - 1)
        def _next_a():
            offa = base + (2 * it + 2) * KDEG
            pltpu.async_copy(src_hbm.at[pl.ds(offa, KDEG)], siA, semA)
            pltpu.async_copy(dst_hbm.at[pl.ds(offa, KDEG)], diA, semA)

        pltpu.make_async_copy(src_hbm.at[pl.ds(base, KDEG)], siB, semB).wait()
        pltpu.make_async_copy(dst_hbm.at[pl.ds(base, KDEG)], diB, semB).wait()
        pltpu.sync_copy(mska_v, acc_sh.at[siB], add=True)
        pltpu.sync_copy(mskb_v, acc_sh.at[diB], add=True)
        return carry

    lax.fori_loop(0, NDPAIR, body, 0)
    offt = base + 2 * NDPAIR * KDEG
    pltpu.sync_copy(src_hbm.at[pl.ds(offt, KDEG)], siA)
    pltpu.sync_copy(dst_hbm.at[pl.ds(offt, KDEG)], diA)
    pltpu.sync_copy(mska_v, acc_sh.at[siA], add=True)
    pltpu.sync_copy(mskb_v, acc_sh.at[diA], add=True)
    plsc.subcore_barrier()
    pltpu.sync_copy(acc_sh.at[pl.ds(s * RA, RA)],
                    out_hbm.at[c, pl.ds(s * RA, RA)])

    @pl.when(s == NS - 1)
    def _otail():
        pltpu.sync_copy(acc_sh.at[pl.ds(TAIL_OFF, TAIL)],
                        out_hbm.at[c, pl.ds(TAIL_OFF, TAIL)])


def _sc_edge_agg_body(src_hbm, dst_hbm, table_hbm, z_hbm, out_hbm,
                      si0, di0, r0, si1, di1, r1, acc_sh,
                      sem0, sem1, semiA, semiB):
    c = lax.axis_index("c")
    s = lax.axis_index("s")
    wid = s * NC + c
    base = wid * EPW
    pltpu.sync_copy(z_hbm.at[pl.ds(0, RA)], acc_sh.at[pl.ds(s * RA, RA)])

    @pl.when(s == NS - 1)
    def _ztail():
        pltpu.sync_copy(z_hbm.at[pl.ds(0, TAIL)],
                        acc_sh.at[pl.ds(TAIL_OFF, TAIL)])

    plsc.subcore_barrier()

    pltpu.sync_copy(src_hbm.at[pl.ds(base, KCH)], si0)
    pltpu.sync_copy(dst_hbm.at[pl.ds(base, KCH)], di0)
    pltpu.async_copy(table_hbm.at[si0], r0, sem0)
    off1 = base + KCH
    pltpu.async_copy(src_hbm.at[pl.ds(off1, KCH)], si1, semiB)
    pltpu.async_copy(dst_hbm.at[pl.ds(off1, KCH)], di1, semiB)

    def body(it, carry):
        pltpu.make_async_copy(src_hbm.at[pl.ds(base, KCH)], si1, semiB).wait()
        pltpu.make_async_copy(dst_hbm.at[pl.ds(base, KCH)], di1, semiB).wait()
        pltpu.async_copy(table_hbm.at[si1], r1, sem1)

        pltpu.make_async_copy(table_hbm.at[si0], r0, sem0).wait()
        pltpu.sync_copy(r0, acc_sh.at[di0], add=True)

        @pl.when(it < NPAIR - 1)
        def _idx_a():
            offa = base + (2 * it + 2) * KCH
            pltpu.async_copy(src_hbm.at[pl.ds(offa, KCH)], si0, semiA)
            pltpu.async_copy(dst_hbm.at[pl.ds(offa, KCH)], di0, semiA)

        pltpu.make_async_copy(table_hbm.at[si1], r1, sem1).wait()

        @pl.when(it < NPAIR - 1)
        def _gather_a():
            pltpu.make_async_copy(src_hbm.at[pl.ds(base, KCH)], si0,
                                  semiA).wait()
            pltpu.make_async_copy(dst_hbm.at[pl.ds(base, KCH)], di0,
                                  semiA).wait()
            pltpu.async_copy(table_hbm.at[si0], r0, sem0)

        pltpu.sync_copy(r1, acc_sh.at[di1], add=True)

        @pl.when(it < NPAIR - 1)
        def _idx_b():
            offb = base + (2 * it + 3) * KCH
            pltpu.async_copy(src_hbm.at[pl.ds(offb, KCH)], si1, semiB)
            pltpu.async_copy(dst_hbm.at[pl.ds(offb, KCH)], di1, semiB)

        return carry

    lax.fori_loop(0, NPAIR, body, 0)
    offt = base + 2 * NPAIR * KCH
    pltpu.sync_copy(src_hbm.at[pl.ds(offt, KCH)], si1)
    pltpu.sync_copy(dst_hbm.at[pl.ds(offt, KCH)], di1)
    pltpu.async_copy(table_hbm.at[si1], r1, sem1).wait()
    pltpu.sync_copy(r1, acc_sh.at[di1], add=True)
    plsc.subcore_barrier()
    pltpu.sync_copy(acc_sh.at[pl.ds(s * RA, RA)],
                    out_hbm.at[c, pl.ds(s * RA, RA)])

    @pl.when(s == NS - 1)
    def _otail():
        pltpu.sync_copy(acc_sh.at[pl.ds(TAIL_OFF, TAIL)],
                        out_hbm.at[c, pl.ds(TAIL_OFF, TAIL)])


@functools.lru_cache(maxsize=None)
def _sc_kernels():
    mesh = plsc.VectorSubcoreMesh(core_axis_name="c", subcore_axis_name="s",
                                  num_cores=NC, num_subcores=NS)
    degrees = pl.kernel(
        _sc_degrees_body,
        out_type=jax.ShapeDtypeStruct((NC, N, H), jnp.float32),
        mesh=mesh,
        scratch_types=[
            pltpu.VMEM((KDEG,), jnp.int32),
            pltpu.VMEM((KDEG,), jnp.int32),
            pltpu.VMEM((KDEG,), jnp.int32),
            pltpu.VMEM((KDEG,), jnp.int32),
            pltpu.VMEM((KDEG, H), jnp.float32),
            pltpu.VMEM((KDEG, H), jnp.float32),
            pltpu.VMEM_SHARED((N, H), jnp.float32),
            pltpu.SemaphoreType.DMA,
            pltpu.SemaphoreType.DMA,
        ],
    )
    edge_agg = pl.kernel(
        _sc_edge_agg_body,
        out_type=jax.ShapeDtypeStruct((NC, N, H), jnp.float32),
        mesh=mesh,
        scratch_types=[
            pltpu.VMEM((KCH,), jnp.int32),
            pltpu.VMEM((KCH,), jnp.int32),
            pltpu.VMEM((KCH, H), jnp.float32),
            pltpu.VMEM((KCH,), jnp.int32),
            pltpu.VMEM((KCH,), jnp.int32),
            pltpu.VMEM((KCH, H), jnp.float32),
            pltpu.VMEM_SHARED((N, H), jnp.float32),
            pltpu.SemaphoreType.DMA,
            pltpu.SemaphoreType.DMA,
            pltpu.SemaphoreType.DMA,
            pltpu.SemaphoreType.DMA,
        ],
    )
    return degrees, edge_agg


_TBN = 2000


def _tc_norms_body(degp_ref, out_ref):
    dp = jnp.sum(degp_ref[...], axis=0)
    deg_s = jnp.sum(dp[:, 0:HH], axis=1) * (1.0 / HH)
    deg_d = jnp.sum(dp[:, HH:H], axis=1) * (1.0 / HH)
    deg = jnp.stack([deg_s, deg_d], axis=0)
    nrm = jnp.where(deg > 0.0, lax.rsqrt(jnp.maximum(deg, 1e-30)), 0.0)
    out_ref[...] = nrm.reshape(1, 2, _TBN)


def _tc_norms(degp):
    nb = N // _TBN
    out = pl.pallas_call(
        _tc_norms_body,
        grid=(nb,),
        in_specs=[pl.BlockSpec((NC, _TBN, H), lambda i: (0, i, 0))],
        out_specs=pl.BlockSpec((1, 2, _TBN), lambda i: (i, 0, 0)),
        out_shape=jax.ShapeDtypeStruct((nb, 2, _TBN), jnp.float32),
    )(degp)
    return jnp.transpose(out, (1, 0, 2)).reshape(2, N)


_TBM = 1000


def _tc_mm_body(x_ref, w_ref, out_ref):
    out_ref[...] = jnp.dot(x_ref[...], w_ref[...],
                           preferred_element_type=jnp.float32)


def _tc_mm(x, w):
    return pl.pallas_call(
        _tc_mm_body,
        grid=(N // _TBM,),
        in_specs=[
            pl.BlockSpec((_TBM, D), lambda i: (i, 0)),
            pl.BlockSpec((D, H), lambda i: (0, 0)),
        ],
        out_specs=pl.BlockSpec((_TBM, H), lambda i: (i, 0)),
        out_shape=jax.ShapeDtypeStruct((N, H), jnp.float32),
    )(x, w)


def _tc_scale_body(xw_ref, nsrc_ref, out_ref):
    out_ref[...] = xw_ref[...] * nsrc_ref[...]


def _tc_scale(xw, nsrc_col):
    return pl.pallas_call(
        _tc_scale_body,
        grid=(N // _TBM,),
        in_specs=[
            pl.BlockSpec((_TBM, H), lambda i: (i, 0)),
            pl.BlockSpec((_TBM, 1), lambda i: (i, 0)),
        ],
        out_specs=pl.BlockSpec((_TBM, H), lambda i: (i, 0)),
        out_shape=jax.ShapeDtypeStruct((N, H), jnp.float32),
    )(xw, nsrc_col)


def _tc_combine_body(parts_ref, ndst_ref, b_ref, w2_ref, nsrc_ref, out_ref):
    agg = (parts_ref[0] + parts_ref[1]) * ndst_ref[...] + b_ref[...]
    h1 = jnp.maximum(agg, 0.0)
    hw = jnp.dot(h1, w2_ref[...], preferred_element_type=jnp.float32)
    out_ref[...] = hw * nsrc_ref[...]


def _tc_combine(parts, ndst_col, b_row, w2, nsrc_col):
    return pl.pallas_call(
        _tc_combine_body,
        grid=(N // _TBM,),
        in_specs=[
            pl.BlockSpec((NC, _TBM, H), lambda i: (0, i, 0)),
            pl.BlockSpec((_TBM, 1), lambda i: (i, 0)),
            pl.BlockSpec((1, H), lambda i: (0, 0)),
            pl.BlockSpec((H, H), lambda i: (0, 0)),
            pl.BlockSpec((_TBM, 1), lambda i: (i, 0)),
        ],
        out_specs=pl.BlockSpec((_TBM, H), lambda i: (i, 0)),
        out_shape=jax.ShapeDtypeStruct((N, H), jnp.float32),
    )(parts, ndst_col, b_row, w2, nsrc_col)


def _tc_proj_body(parts_ref, ndst_ref, b2_ref, wihT_ref, bsum_ref, out_ref):
    agg = (parts_ref[0] + parts_ref[1]) * ndst_ref[...] + b2_ref[...]
    h2b = jnp.maximum(agg, 0.0)
    out_ref[...] = jnp.dot(h2b, wihT_ref[...],
                           preferred_element_type=jnp.float32) + bsum_ref[...]


def _tc_proj(parts, ndst_col, b2_row, wihT, bsum_row):
    return pl.pallas_call(
        _tc_proj_body,
        grid=(N // _TBM,),
        in_specs=[
            pl.BlockSpec((NC, _TBM, H), lambda i: (0, i, 0)),
            pl.BlockSpec((_TBM, 1), lambda i: (i, 0)),
            pl.BlockSpec((1, H), lambda i: (0, 0)),
            pl.BlockSpec((H, G4), lambda i: (0, 0)),
            pl.BlockSpec((1, G4), lambda i: (0, 0)),
        ],
        out_specs=pl.BlockSpec((_TBM, G4), lambda i: (i, 0)),
        out_shape=jax.ShapeDtypeStruct((N, G4), jnp.float32),
    )(parts, ndst_col, b2_row, wihT, bsum_row)


_TBL = 2000
_NGB = N // _TBL
_UNR = 16
H3 = 3 * H


def _tc_lstm_body(p_ref, whhT_ref, fcWT_ref, fcb_ref, out_ref, h_scr, c_scr):
    pid = pl.program_id(0)

    @pl.when(pid == 0)
    def _init():
        h_scr[...] = jnp.zeros_like(h_scr)
        c_scr[...] = jnp.zeros_like(c_scr)

    def step8(j, carry):
        h, cc = carry
        p8 = p_ref[pl.ds(j * _UNR, _UNR), :]
        for k in range(_UNR):
            u = jnp.sum(whhT_ref[...] * h.reshape(H, 1), axis=0,
                        keepdims=True)
            gates = p8[k:k + 1, :] + u
            s = jnp.tanh(gates[:, 0:H3]) * 0.5 + 0.5
            g_g = jnp.tanh(gates[:, H3:G4])
            cc = s[:, H:2 * H] * cc + s[:, 0:H] * g_g
            h = s[:, 2 * H:H3] * jnp.tanh(cc)
        return (h, cc)

    hN, cN = lax.fori_loop(0, _TBL // _UNR, step8, (h_scr[...], c_scr[...]))
    h_scr[...] = hN
    c_scr[...] = cN

    @pl.when(pid == _NGB - 1)
    def _fin():
        out_ref[...] = jnp.dot(hN, fcWT_ref[...],
                               preferred_element_type=jnp.float32) + fcb_ref[...]


def _tc_lstm(p_all, whhT, fcWT, fcb_row):
    return pl.pallas_call(
        _tc_lstm_body,
        grid=(_NGB,),
        in_specs=[
            pl.BlockSpec((_TBL, G4), lambda i: (i, 0)),
            pl.BlockSpec((H, G4), lambda i: (0, 0)),
            pl.BlockSpec((H, CO), lambda i: (0, 0)),
            pl.BlockSpec((1, CO), lambda i: (0, 0)),
        ],
        out_specs=pl.BlockSpec((1, CO), lambda i: (0, 0)),
        out_shape=jax.ShapeDtypeStruct((1, CO), jnp.float32),
        scratch_shapes=[
            pltpu.VMEM((1, H), jnp.float32),
            pltpu.VMEM((1, H), jnp.float32),
        ],
        compiler_params=pltpu.CompilerParams(
            dimension_semantics=("arbitrary",)),
    )(p_all, whhT, fcWT, fcb_row)


def kernel(features, edge_index, W1, b1, W2, b2, W_ih, W_hh, b_ih, b_hh,
           fc_W, fc_b):
    src = edge_index[0]
    dst = edge_index[1]
    zrows = jnp.zeros((RA, H), jnp.float32)
    lane = lax.broadcasted_iota(jnp.int32, (KDEG, H), 1)
    mska = jnp.where(lane < HH, 1.0, 0.0).astype(jnp.float32)
    mskb = 1.0 - mska
    _sc_degrees, _sc_edge_agg = _sc_kernels()

    xw = _tc_mm(features, W1)
    degp = _sc_degrees(src, dst, zrows, mska, mskb)
    norms = _tc_norms(degp)
    nsrc_col = norms[0].reshape(N, 1)
    ndst_col = norms[1].reshape(N, 1)

    hs1 = _tc_scale(xw, nsrc_col)
    parts1 = _sc_edge_agg(src, dst, hs1, zrows)
    hs2 = _tc_combine(parts1, ndst_col, b1.reshape(1, H), W2, nsrc_col)
    parts2 = _sc_edge_agg(src, dst, hs2, zrows)

    perm = jnp.concatenate([jnp.arange(0, 2 * H), jnp.arange(3 * H, 4 * H),
                            jnp.arange(2 * H, 3 * H)])
    scal = jnp.where(jnp.arange(G4) < H3, 0.5, 1.0)[:, None]
    wihT_p = (W_ih[perm] * scal).T
    whhT_p = (W_hh[perm] * scal).T
    bsum_p = ((b_ih + b_hh)[perm] * scal[:, 0]).reshape(1, G4)

    p_all = _tc_proj(parts2, ndst_col, b2.reshape(1, H), wihT_p, bsum_p)
    out = _tc_lstm(p_all, whhT_p, fc_W.T, fc_b.reshape(1, CO))
    return out


# speedup vs baseline: 1.0034x; 1.0034x over previous
import functools

import jax
import jax.numpy as jnp
from jax import lax
from jax.experimental import pallas as pl
from jax.experimental.pallas import tpu as pltpu
from jax.experimental.pallas import tpu_sc as plsc

N = 10000
D = 128
H = 128
G4 = 4 * H
CO = 64
E = 320000

NC = 2
NS = 16
NW = NC * NS
EPW = E // NW
KCH = 80
NCHUNK = EPW // KCH
NPAIR = NCHUNK // 2
KDEG = 80
NDCH = EPW // KDEG
NDPAIR = NDCH // 2
HH = H // 2
RA = 624
TAIL = N - NS * RA
TAIL_OFF = NS * RA

def _sc_degrees_body(src_hbm, dst_hbm, z_hbm, mska_hbm, mskb_hbm, out_hbm,
                     siA, diA, siB, diB, mska_v, mskb_v, acc_sh, semA, semB):
    c = lax.axis_index("c")
    s = lax.axis_index("s")
    wid = s * NC + c
    base = wid * EPW
    pltpu.sync_copy(mska_hbm, mska_v)
    pltpu.sync_copy(mskb_hbm, mskb_v)
    pltpu.sync_copy(z_hbm.at[pl.ds(0, RA)], acc_sh.at[pl.ds(s * RA, RA)])

    @pl.when(s == NS - 1)
    def _ztail():
        pltpu.sync_copy(z_hbm.at[pl.ds(0, TAIL)],
                        acc_sh.at[pl.ds(TAIL_OFF, TAIL)])

    plsc.subcore_barrier()

    pltpu.async_copy(src_hbm.at[pl.ds(base, KDEG)], siA, semA)
    pltpu.async_copy(dst_hbm.at[pl.ds(base, KDEG)], diA, semA)

    def body(it, carry):
        offb = base + (2 * it + 1) * KDEG
        pltpu.async_copy(src_hbm.at[pl.ds(offb, KDEG)], siB, semB)
        pltpu.async_copy(dst_hbm.at[pl.ds(offb, KDEG)], diB, semB)

        pltpu.make_async_copy(src_hbm.at[pl.ds(base, KDEG)], siA, semA).wait()
        pltpu.make_async_copy(dst_hbm.at[pl.ds(base, KDEG)], diA, semA).wait()
        pltpu.sync_copy(mska_v, acc_sh.at[siA], add=True)
        pltpu.sync_copy(mskb_v, acc_sh.at[diA], add=True)

        @pl.when(it < NDPAIR - 1)
        def _next_a():
            offa = base + (2 * it + 2) * KDEG
            pltpu.async_copy(src_hbm.at[pl.ds(offa, KDEG)], siA, semA)
            pltpu.async_copy(dst_hbm.at[pl.ds(offa, KDEG)], diA, semA)

        pltpu.make_async_copy(src_hbm.at[pl.ds(base, KDEG)], siB, semB).wait()
        pltpu.make_async_copy(dst_hbm.at[pl.ds(base, KDEG)], diB, semB).wait()
        pltpu.sync_copy(mska_v, acc_sh.at[siB], add=True)
        pltpu.sync_copy(mskb_v, acc_sh.at[diB], add=True)
        return carry

    lax.fori_loop(0, NDPAIR, body, 0)
    offt = base + 2 * NDPAIR * KDEG
    pltpu.sync_copy(src_hbm.at[pl.ds(offt, KDEG)], siA)
    pltpu.sync_copy(dst_hbm.at[pl.ds(offt, KDEG)], diA)
    pltpu.sync_copy(mska_v, acc_sh.at[siA], add=True)
    pltpu.sync_copy(mskb_v, acc_sh.at[diA], add=True)
    plsc.subcore_barrier()
    pltpu.sync_copy(acc_sh.at[pl.ds(s * RA, RA)],
                    out_hbm.at[c, pl.ds(s * RA, RA)])

    @pl.when(s == NS - 1)
    def _otail():
        pltpu.sync_copy(acc_sh.at[pl.ds(TAIL_OFF, TAIL)],
                        out_hbm.at[c, pl.ds(TAIL_OFF, TAIL)])


def _sc_edge_agg_body(src_hbm, dst_hbm, table_hbm, z_hbm, out_hbm,
                      si0, di0, r0, si1, di1, r1, acc_sh,
                      sem0, sem1, semiA, semiB, semS):
    c = lax.axis_index("c")
    s = lax.axis_index("s")
    wid = s * NC + c
    base = wid * EPW
    pltpu.sync_copy(z_hbm.at[pl.ds(0, RA)], acc_sh.at[pl.ds(s * RA, RA)])

    @pl.when(s == NS - 1)
    def _ztail():
        pltpu.sync_copy(z_hbm.at[pl.ds(0, TAIL)],
                        acc_sh.at[pl.ds(TAIL_OFF, TAIL)])

    plsc.subcore_barrier()

    pltpu.sync_copy(src_hbm.at[pl.ds(base, KCH)], si0)
    pltpu.sync_copy(dst_hbm.at[pl.ds(base, KCH)], di0)
    pltpu.async_copy(table_hbm.at[si0], r0, sem0)
    off1 = base + KCH
    pltpu.async_copy(src_hbm.at[pl.ds(off1, KCH)], si1, semiB)
    pltpu.async_copy(dst_hbm.at[pl.ds(off1, KCH)], di1, semiB)

    def body(it, carry):
        pltpu.make_async_copy(src_hbm.at[pl.ds(base, KCH)], si1, semiB).wait()
        pltpu.make_async_copy(dst_hbm.at[pl.ds(base, KCH)], di1, semiB).wait()
        pltpu.async_copy(table_hbm.at[si1], r1, sem1)

        pltpu.make_async_copy(table_hbm.at[si0], r0, sem0).wait()
        pltpu.sync_copy(r0, acc_sh.at[di0], add=True)

        @pl.when(it < NPAIR - 1)
        def _idx_a():
            offa = base + (2 * it + 2) * KCH
            pltpu.async_copy(src_hbm.at[pl.ds(offa, KCH)], si0, semiA)
            pltpu.async_copy(dst_hbm.at[pl.ds(offa, KCH)], di0, semiA)

        pltpu.make_async_copy(table_hbm.at[si1], r1, sem1).wait()
        pltpu.async_copy(r1, acc_sh.at[di1], semS, add=True)

        @pl.when(it < NPAIR - 1)
        def _gather_a():
            pltpu.make_async_copy(src_hbm.at[pl.ds(base, KCH)], si0,
                                  semiA).wait()
            pltpu.make_async_copy(dst_hbm.at[pl.ds(base, KCH)], di0,
                                  semiA).wait()
            pltpu.async_copy(table_hbm.at[si0], r0, sem0)

        pltpu.make_async_copy(z_hbm.at[pl.ds(0, KCH)], r1, semS).wait()

        @pl.when(it < NPAIR - 1)
        def _idx_b():
            offb = base + (2 * it + 3) * KCH
            pltpu.async_copy(src_hbm.at[pl.ds(offb, KCH)], si1, semiB)
            pltpu.async_copy(dst_hbm.at[pl.ds(offb, KCH)], di1, semiB)

        return carry

    lax.fori_loop(0, NPAIR, body, 0)
    offt = base + 2 * NPAIR * KCH
    pltpu.sync_copy(src_hbm.at[pl.ds(offt, KCH)], si1)
    pltpu.sync_copy(dst_hbm.at[pl.ds(offt, KCH)], di1)
    pltpu.async_copy(table_hbm.at[si1], r1, sem1).wait()
    pltpu.sync_copy(r1, acc_sh.at[di1], add=True)
    plsc.subcore_barrier()
    pltpu.sync_copy(acc_sh.at[pl.ds(s * RA, RA)],
                    out_hbm.at[c, pl.ds(s * RA, RA)])

    @pl.when(s == NS - 1)
    def _otail():
        pltpu.sync_copy(acc_sh.at[pl.ds(TAIL_OFF, TAIL)],
                        out_hbm.at[c, pl.ds(TAIL_OFF, TAIL)])


@functools.lru_cache(maxsize=None)
def _sc_kernels():
    mesh = plsc.VectorSubcoreMesh(core_axis_name="c", subcore_axis_name="s",
                                  num_cores=NC, num_subcores=NS)
    degrees = pl.kernel(
        _sc_degrees_body,
        out_type=jax.ShapeDtypeStruct((NC, N, H), jnp.float32),
        mesh=mesh,
        scratch_types=[
            pltpu.VMEM((KDEG,), jnp.int32),
            pltpu.VMEM((KDEG,), jnp.int32),
            pltpu.VMEM((KDEG,), jnp.int32),
            pltpu.VMEM((KDEG,), jnp.int32),
            pltpu.VMEM((KDEG, H), jnp.float32),
            pltpu.VMEM((KDEG, H), jnp.float32),
            pltpu.VMEM_SHARED((N, H), jnp.float32),
            pltpu.SemaphoreType.DMA,
            pltpu.SemaphoreType.DMA,
        ],
    )
    edge_agg = pl.kernel(
        _sc_edge_agg_body,
        out_type=jax.ShapeDtypeStruct((NC, N, H), jnp.float32),
        mesh=mesh,
        scratch_types=[
            pltpu.VMEM((KCH,), jnp.int32),
            pltpu.VMEM((KCH,), jnp.int32),
            pltpu.VMEM((KCH, H), jnp.float32),
            pltpu.VMEM((KCH,), jnp.int32),
            pltpu.VMEM((KCH,), jnp.int32),
            pltpu.VMEM((KCH, H), jnp.float32),
            pltpu.VMEM_SHARED((N, H), jnp.float32),
            pltpu.SemaphoreType.DMA,
            pltpu.SemaphoreType.DMA,
            pltpu.SemaphoreType.DMA,
            pltpu.SemaphoreType.DMA,
            pltpu.SemaphoreType.DMA,
        ],
    )
    return degrees, edge_agg


_TBN = 2000


def _tc_norms_body(degp_ref, out_ref):
    dp = jnp.sum(degp_ref[...], axis=0)
    deg_s = jnp.sum(dp[:, 0:HH], axis=1) * (1.0 / HH)
    deg_d = jnp.sum(dp[:, HH:H], axis=1) * (1.0 / HH)
    deg = jnp.stack([deg_s, deg_d], axis=0)
    nrm = jnp.where(deg > 0.0, lax.rsqrt(jnp.maximum(deg, 1e-30)), 0.0)
    out_ref[...] = nrm.reshape(1, 2, _TBN)


def _tc_norms(degp):
    nb = N // _TBN
    out = pl.pallas_call(
        _tc_norms_body,
        grid=(nb,),
        in_specs=[pl.BlockSpec((NC, _TBN, H), lambda i: (0, i, 0))],
        out_specs=pl.BlockSpec((1, 2, _TBN), lambda i: (i, 0, 0)),
        out_shape=jax.ShapeDtypeStruct((nb, 2, _TBN), jnp.float32),
    )(degp)
    return jnp.transpose(out, (1, 0, 2)).reshape(2, N)


_TBM = 1000


def _tc_mm_body(x_ref, w_ref, out_ref):
    out_ref[...] = jnp.dot(x_ref[...], w_ref[...],
                           preferred_element_type=jnp.float32)


def _tc_mm(x, w):
    return pl.pallas_call(
        _tc_mm_body,
        grid=(N // _TBM,),
        in_specs=[
            pl.BlockSpec((_TBM, D), lambda i: (i, 0)),
            pl.BlockSpec((D, H), lambda i: (0, 0)),
        ],
        out_specs=pl.BlockSpec((_TBM, H), lambda i: (i, 0)),
        out_shape=jax.ShapeDtypeStruct((N, H), jnp.float32),
    )(x, w)


def _tc_scale_body(xw_ref, nsrc_ref, out_ref):
    out_ref[...] = xw_ref[...] * nsrc_ref[...]


def _tc_scale(xw, nsrc_col):
    return pl.pallas_call(
        _tc_scale_body,
        grid=(N // _TBM,),
        in_specs=[
            pl.BlockSpec((_TBM, H), lambda i: (i, 0)),
            pl.BlockSpec((_TBM, 1), lambda i: (i, 0)),
        ],
        out_specs=pl.BlockSpec((_TBM, H), lambda i: (i, 0)),
        out_shape=jax.ShapeDtypeStruct((N, H), jnp.float32),
    )(xw, nsrc_col)


def _tc_combine_body(parts_ref, ndst_ref, b_ref, w2_ref, nsrc_ref, out_ref):
    agg = (parts_ref[0] + parts_ref[1]) * ndst_ref[...] + b_ref[...]
    h1 = jnp.maximum(agg, 0.0)
    hw = jnp.dot(h1, w2_ref[...], preferred_element_type=jnp.float32)
    out_ref[...] = hw * nsrc_ref[...]


def _tc_combine(parts, ndst_col, b_row, w2, nsrc_col):
    return pl.pallas_call(
        _tc_combine_body,
        grid=(N // _TBM,),
        in_specs=[
            pl.BlockSpec((NC, _TBM, H), lambda i: (0, i, 0)),
            pl.BlockSpec((_TBM, 1), lambda i: (i, 0)),
            pl.BlockSpec((1, H), lambda i: (0, 0)),
            pl.BlockSpec((H, H), lambda i: (0, 0)),
            pl.BlockSpec((_TBM, 1), lambda i: (i, 0)),
        ],
        out_specs=pl.BlockSpec((_TBM, H), lambda i: (i, 0)),
        out_shape=jax.ShapeDtypeStruct((N, H), jnp.float32),
    )(parts, ndst_col, b_row, w2, nsrc_col)


def _tc_proj_body(parts_ref, ndst_ref, b2_ref, wihT_ref, bsum_ref, out_ref):
    agg = (parts_ref[0] + parts_ref[1]) * ndst_ref[...] + b2_ref[...]
    h2b = jnp.maximum(agg, 0.0)
    out_ref[...] = jnp.dot(h2b, wihT_ref[...],
                           preferred_element_type=jnp.float32) + bsum_ref[...]


def _tc_proj(parts, ndst_col, b2_row, wihT, bsum_row):
    return pl.pallas_call(
        _tc_proj_body,
        grid=(N // _TBM,),
        in_specs=[
            pl.BlockSpec((NC, _TBM, H), lambda i: (0, i, 0)),
            pl.BlockSpec((_TBM, 1), lambda i: (i, 0)),
            pl.BlockSpec((1, H), lambda i: (0, 0)),
            pl.BlockSpec((H, G4), lambda i: (0, 0)),
            pl.BlockSpec((1, G4), lambda i: (0, 0)),
        ],
        out_specs=pl.BlockSpec((_TBM, G4), lambda i: (i, 0)),
        out_shape=jax.ShapeDtypeStruct((N, G4), jnp.float32),
    )(parts, ndst_col, b2_row, wihT, bsum_row)


_TBL = 2000
_NGB = N // _TBL
_UNR = 16
H3 = 3 * H


def _tc_lstm_body(p_ref, whhT_ref, fcWT_ref, fcb_ref, out_ref, h_scr, c_scr):
    pid = pl.program_id(0)

    @pl.when(pid == 0)
    def _init():
        h_scr[...] = jnp.zeros_like(h_scr)
        c_scr[...] = jnp.zeros_like(c_scr)

    def step8(j, carry):
        h, cc = carry
        p8 = p_ref[pl.ds(j * _UNR, _UNR), :]
        for k in range(_UNR):
            u = jnp.sum(whhT_ref[...] * h.reshape(H, 1), axis=0,
                        keepdims=True)
            gates = p8[k:k + 1, :] + u
            s = jnp.tanh(gates[:, 0:H3]) * 0.5 + 0.5
            g_g = jnp.tanh(gates[:, H3:G4])
            cc = s[:, H:2 * H] * cc + s[:, 0:H] * g_g
            h = s[:, 2 * H:H3] * jnp.tanh(cc)
        return (h, cc)

    hN, cN = lax.fori_loop(0, _TBL // _UNR, step8, (h_scr[...], c_scr[...]))
    h_scr[...] = hN
    c_scr[...] = cN

    @pl.when(pid == _NGB - 1)
    def _fin():
        out_ref[...] = jnp.dot(hN, fcWT_ref[...],
                               preferred_element_type=jnp.float32) + fcb_ref[...]


def _tc_lstm(p_all, whhT, fcWT, fcb_row):
    return pl.pallas_call(
        _tc_lstm_body,
        grid=(_NGB,),
        in_specs=[
            pl.BlockSpec((_TBL, G4), lambda i: (i, 0)),
            pl.BlockSpec((H, G4), lambda i: (0, 0)),
            pl.BlockSpec((H, CO), lambda i: (0, 0)),
            pl.BlockSpec((1, CO), lambda i: (0, 0)),
        ],
        out_specs=pl.BlockSpec((1, CO), lambda i: (0, 0)),
        out_shape=jax.ShapeDtypeStruct((1, CO), jnp.float32),
        scratch_shapes=[
            pltpu.VMEM((1, H), jnp.float32),
            pltpu.VMEM((1, H), jnp.float32),
        ],
        compiler_params=pltpu.CompilerParams(
            dimension_semantics=("arbitrary",)),
    )(p_all, whhT, fcWT, fcb_row)


def kernel(features, edge_index, W1, b1, W2, b2, W_ih, W_hh, b_ih, b_hh,
           fc_W, fc_b):
    src = edge_index[0]
    dst = edge_index[1]
    zrows = jnp.zeros((RA, H), jnp.float32)
    lane = lax.broadcasted_iota(jnp.int32, (KDEG, H), 1)
    mska = jnp.where(lane < HH, 1.0, 0.0).astype(jnp.float32)
    mskb = 1.0 - mska
    _sc_degrees, _sc_edge_agg = _sc_kernels()

    xw = _tc_mm(features, W1)
    degp = _sc_degrees(src, dst, zrows, mska, mskb)
    norms = _tc_norms(degp)
    nsrc_col = norms[0].reshape(N, 1)
    ndst_col = norms[1].reshape(N, 1)

    hs1 = _tc_scale(xw, nsrc_col)
    parts1 = _sc_edge_agg(src, dst, hs1, zrows)
    hs2 = _tc_combine(parts1, ndst_col, b1.reshape(1, H), W2, nsrc_col)
    parts2 = _sc_edge_agg(src, dst, hs2, zrows)

    perm = jnp.concatenate([jnp.arange(0, 2 * H), jnp.arange(3 * H, 4 * H),
                            jnp.arange(2 * H, 3 * H)])
    scal = jnp.where(jnp.arange(G4) < H3, 0.5, 1.0)[:, None]
    wihT_p = (W_ih[perm] * scal).T
    whhT_p = (W_hh[perm] * scal).T
    bsum_p = ((b_ih + b_hh)[perm] * scal[:, 0]).reshape(1, G4)

    p_all = _tc_proj(parts2, ndst_col, b2.reshape(1, H), wihT_p, bsum_p)
    out = _tc_lstm(p_all, whhT_p, fc_W.T, fc_b.reshape(1, CO))
    return out
